# baseline (speedup 1.0000x reference)
import jax
import jax.numpy as jnp
from jax.experimental import pallas as pl
from jax.experimental.pallas import tpu as pltpu

_XH = 16
_ZH = 8


def _body(x_hbm, y_ref, z_hbm, w_ref, xo_hbm, zo_hbm, xt, zt,
          sx, sz, sxi, szi, sxo, szo):
    nx = x_hbm.shape[0]
    nz = z_hbm.shape[0]
    big_x = pltpu.make_async_copy(x_hbm.at[pl.ds(_XH, nx - _XH), :],
                                  xo_hbm.at[pl.ds(_XH, nx - _XH), :], sx)
    big_x.start()
    big_z = pltpu.make_async_copy(z_hbm.at[pl.ds(_ZH, nz - _ZH), :],
                                  zo_hbm.at[pl.ds(_ZH, nz - _ZH), :], sz)
    big_z.start()
    cxi = pltpu.make_async_copy(x_hbm.at[pl.ds(0, _XH), :], xt, sxi)
    cxi.start()
    czi = pltpu.make_async_copy(z_hbm.at[pl.ds(0, _ZH), :], zt, szi)
    czi.start()

    cxi.wait()
    r = jax.lax.broadcasted_iota(jnp.int32, (_XH, 256), 0)
    b = xt[...]
    b = jnp.where(r == 10, y_ref[0, :][None, :], b)
    b = jnp.where(r == 2, y_ref[1, :][None, :], b)
    b = jnp.where(r == 1, jnp.float32(45.0), b)
    xt[...] = b

    czi.wait()
    rz = jax.lax.broadcasted_iota(jnp.int32, (_ZH, 1024), 0)
    cz = jax.lax.broadcasted_iota(jnp.int32, (_ZH, 1024), 1)
    add = (w_ref[0] * ((rz == 1) & (cz == 3)).astype(jnp.float32)
           + w_ref[1] * ((rz == 0) & (cz == 2)).astype(jnp.float32)
           + w_ref[2] * ((rz == 0) & (cz == 1)).astype(jnp.float32))
    zt[...] = zt[...] + add

    cxo = pltpu.make_async_copy(xt, xo_hbm.at[pl.ds(0, _XH), :], sxo)
    cxo.start()
    czo = pltpu.make_async_copy(zt, zo_hbm.at[pl.ds(0, _ZH), :], szo)
    czo.start()
    cxo.wait()
    czo.wait()
    big_x.wait()
    big_z.wait()


def kernel(x, y, z, w):
    xo, zo = pl.pallas_call(
        _body,
        in_specs=[
            pl.BlockSpec(memory_space=pl.ANY),
            pl.BlockSpec(memory_space=pltpu.VMEM),
            pl.BlockSpec(memory_space=pl.ANY),
            pl.BlockSpec(memory_space=pltpu.SMEM),
        ],
        out_specs=[
            pl.BlockSpec(memory_space=pl.ANY),
            pl.BlockSpec(memory_space=pl.ANY),
        ],
        out_shape=[
            jax.ShapeDtypeStruct(x.shape, x.dtype),
            jax.ShapeDtypeStruct(z.shape, z.dtype),
        ],
        scratch_shapes=[
            pltpu.VMEM((_XH, 256), jnp.float32),
            pltpu.VMEM((_ZH, 1024), jnp.float32),
            pltpu.SemaphoreType.DMA,
            pltpu.SemaphoreType.DMA,
            pltpu.SemaphoreType.DMA,
            pltpu.SemaphoreType.DMA,
            pltpu.SemaphoreType.DMA,
            pltpu.SemaphoreType.DMA,
        ],
    )(x, y, z, w)
    return (xo, zo)
